# baseline (speedup 1.0000x reference)
import functools

import numpy as np
import jax
import jax.numpy as jnp
from jax import lax
from jax.experimental import pallas as pl
from jax.experimental.pallas import tpu as pltpu
from jax.experimental.pallas import tpu_sc as plsc

NUM_EMB = 1000000
EMB_DIM = 64
HASHED_SIZE = 100000
BATCH = 4096
BAG = 50

NUM_WORKERS = 32
ROWS_PER_W = BATCH * BAG // NUM_WORKERS
CHUNK = 2 * BAG
CHUNKS = ROWS_PER_W // CHUNK
BAGS_PER_W = BATCH // NUM_WORKERS
NBUF = 8
GROUPS = CHUNKS // NBUF

_SLOT_NP = (
    (np.arange(ROWS_PER_W, dtype=np.int32) // BAG)[None, :]
    + (np.arange(16, dtype=np.int32) * BAGS_PER_W)[:, None]
).reshape(16, CHUNKS, CHUNK)


def _make_kernel():
    mesh = plsc.VectorSubcoreMesh(core_axis_name="c", subcore_axis_name="s")

    @functools.partial(
        pl.kernel,
        mesh=mesh,
        out_type=jax.ShapeDtypeStruct((BATCH, EMB_DIM), jnp.float32),
        scratch_types=[
            pltpu.VMEM((CHUNKS, CHUNK), jnp.int32),
            pltpu.VMEM((CHUNKS, CHUNK), jnp.int32),
            pltpu.VMEM((NBUF, CHUNK), jnp.int32),
            pltpu.VMEM((NBUF, CHUNK, EMB_DIM), jnp.float32),
            pltpu.VMEM((BAGS_PER_W, EMB_DIM), jnp.float32),
            pltpu.VMEM_SHARED((16 * BAGS_PER_W, EMB_DIM), jnp.float32),
        ] + [pltpu.SemaphoreType.DMA] * (3 * NBUF),
        compiler_params=pltpu.CompilerParams(use_tc_tiling_on_sc=False),
    )
    def bag_kernel(x_hbm, wi_hbm, hw_hbm, slot_hbm, out_hbm,
                   x_v, slot_v, ids_v, rows_v, stage_v, acc_sh, *sems):
        sem_ids = sems[:NBUF]
        sem_rows = sems[NBUF:2 * NBUF]
        sem_sc = sems[2 * NBUF:]
        cid = lax.axis_index("c")
        sid = lax.axis_index("s")
        wid = sid * 2 + cid

        pltpu.sync_copy(x_hbm.at[wid], x_v)

        def fire_ids(c, b):
            pltpu.async_copy(wi_hbm.at[x_v.at[c]], ids_v.at[b], sem_ids[b])

        def wait_ids(c, b):
            pltpu.make_async_copy(
                wi_hbm.at[x_v.at[c]], ids_v.at[b], sem_ids[b]
            ).wait()

        def fire_rows(c, b):
            pltpu.async_copy(hw_hbm.at[ids_v.at[b]], rows_v.at[b], sem_rows[b])

        def wait_rows(c, b):
            pltpu.make_async_copy(
                hw_hbm.at[ids_v.at[b]], rows_v.at[b], sem_rows[b]
            ).wait()

        def fire_scatter(c, b):
            pltpu.async_copy(rows_v.at[b], acc_sh.at[slot_v.at[c]], sem_sc[b],
                             add=True)

        def wait_scatter(c, b):
            pltpu.make_async_copy(
                rows_v.at[b], acc_sh.at[slot_v.at[c]], sem_sc[b]
            ).wait()

        for b in range(NBUF):
            fire_ids(b, b)

        pltpu.sync_copy(slot_hbm.at[sid], slot_v)

        zeros = jnp.zeros((16,), jnp.float32)

        def zero_body(r, carry):
            for k in range(EMB_DIM // 16):
                stage_v[r, pl.ds(k * 16, 16)] = zeros
            return carry

        lax.fori_loop(0, BAGS_PER_W, zero_body, 0)
        pltpu.sync_copy(stage_v, acc_sh.at[pl.ds(sid * BAGS_PER_W, BAGS_PER_W)])

        def group_body(g, carry):
            for u in range(NBUF):
                c = g * NBUF + u
                pu = (u - 1) % NBUF

                @pl.when(c >= NBUF)
                def _free_rows_buf():
                    wait_scatter(c - NBUF, u)

                wait_ids(c, u)
                fire_rows(c, u)

                @pl.when(c >= 1)
                def _retire_prev():
                    wait_rows(c - 1, pu)
                    fire_scatter(c - 1, pu)

                    @pl.when(c - 1 + NBUF < CHUNKS)
                    def _next_ids():
                        fire_ids(c - 1 + NBUF, pu)

            return carry

        lax.fori_loop(0, GROUPS, group_body, 0)

        last = CHUNKS - 1
        wait_rows(last, last % NBUF)
        fire_scatter(last, last % NBUF)
        for c in range(CHUNKS - NBUF, CHUNKS):
            wait_scatter(c, c % NBUF)

        pltpu.sync_copy(
            acc_sh.at[pl.ds(sid * BAGS_PER_W, BAGS_PER_W)],
            out_hbm.at[pl.ds(wid * BAGS_PER_W, BAGS_PER_W)],
        )

    return bag_kernel


_BAG_KERNEL = _make_kernel()


@jax.jit
def kernel(x, hashed_weight, weight_idx):
    xr = x.reshape(NUM_WORKERS, CHUNKS, CHUNK)
    slot = jnp.asarray(_SLOT_NP)
    return _BAG_KERNEL(xr, weight_idx, hashed_weight, slot)


# speedup vs baseline: 1.0020x; 1.0020x over previous
import functools

import numpy as np
import jax
import jax.numpy as jnp
from jax import lax
from jax.experimental import pallas as pl
from jax.experimental.pallas import tpu as pltpu
from jax.experimental.pallas import tpu_sc as plsc

NUM_EMB = 1000000
EMB_DIM = 64
HASHED_SIZE = 100000
BATCH = 4096
BAG = 50

NUM_WORKERS = 32
ROWS_PER_W = BATCH * BAG // NUM_WORKERS
CHUNK = 2 * BAG
CHUNKS = ROWS_PER_W // CHUNK
ICHUNK = 128
ICHUNKS = ROWS_PER_W // ICHUNK
BAGS_PER_W = BATCH // NUM_WORKERS
NBUF = 4

_SLOT_NP = (
    (np.arange(ROWS_PER_W, dtype=np.int32) // BAG)[None, :]
    + (np.arange(16, dtype=np.int32) * BAGS_PER_W)[:, None]
).reshape(16, CHUNKS, CHUNK)


def _make_ids_kernel():
    mesh = plsc.VectorSubcoreMesh(core_axis_name="c", subcore_axis_name="s")

    @functools.partial(
        pl.kernel,
        mesh=mesh,
        out_type=jax.ShapeDtypeStruct((NUM_WORKERS, ICHUNKS, ICHUNK), jnp.int32),
        scratch_types=[
            pltpu.VMEM((ICHUNKS, ICHUNK), jnp.int32),
            pltpu.VMEM((ICHUNKS, ICHUNK), jnp.int32),
        ] + [pltpu.SemaphoreType.DMA] * NBUF,
        compiler_params=pltpu.CompilerParams(use_tc_tiling_on_sc=False),
    )
    def ids_kernel(x_hbm, wi_hbm, ids_hbm, x_v, ids_v, *sems):
        wid = lax.axis_index("s") * 2 + lax.axis_index("c")

        pltpu.sync_copy(x_hbm.at[wid], x_v)

        def fire(c, b):
            pltpu.async_copy(wi_hbm.at[x_v.at[c]], ids_v.at[c], sems[b])

        def wait(c, b):
            pltpu.make_async_copy(
                wi_hbm.at[x_v.at[c]], ids_v.at[c], sems[b]
            ).wait()

        for b in range(NBUF):
            fire(b, b)

        def body(c, carry):
            for b in range(NBUF):
                cc = c * NBUF + b
                wait(cc, b)

                @pl.when(cc + NBUF < ICHUNKS)
                def _next():
                    fire(cc + NBUF, b)

            return carry

        lax.fori_loop(0, ICHUNKS // NBUF, body, 0)
        for cc in range(ICHUNKS - ICHUNKS % NBUF, ICHUNKS):
            wait(cc, cc % NBUF)

        pltpu.sync_copy(ids_v, ids_hbm.at[wid])

    return ids_kernel


def _make_bag_kernel():
    mesh = plsc.VectorSubcoreMesh(core_axis_name="c", subcore_axis_name="s")

    @functools.partial(
        pl.kernel,
        mesh=mesh,
        out_type=jax.ShapeDtypeStruct((BATCH, EMB_DIM), jnp.float32),
        scratch_types=[
            pltpu.VMEM((CHUNKS, CHUNK), jnp.int32),
            pltpu.VMEM((CHUNKS, CHUNK), jnp.int32),
            pltpu.VMEM((NBUF, CHUNK, EMB_DIM), jnp.float32),
            pltpu.VMEM((BAGS_PER_W, EMB_DIM), jnp.float32),
            pltpu.VMEM_SHARED((16 * BAGS_PER_W, EMB_DIM), jnp.float32),
        ] + [pltpu.SemaphoreType.DMA] * (2 * NBUF),
        compiler_params=pltpu.CompilerParams(use_tc_tiling_on_sc=False),
    )
    def bag_kernel(ids_hbm, hw_hbm, slot_hbm, out_hbm,
                   ids_v, slot_v, rows_v, stage_v, acc_sh, *sems):
        sem_rows = sems[:NBUF]
        sem_sc = sems[NBUF:]
        cid = lax.axis_index("c")
        sid = lax.axis_index("s")
        wid = sid * 2 + cid

        pltpu.sync_copy(ids_hbm.at[wid], ids_v)
        pltpu.sync_copy(slot_hbm.at[sid], slot_v)

        zeros = jnp.zeros((16,), jnp.float32)

        def zero_body(r, carry):
            for k in range(EMB_DIM // 16):
                stage_v[r, pl.ds(k * 16, 16)] = zeros
            return carry

        lax.fori_loop(0, BAGS_PER_W, zero_body, 0)
        pltpu.sync_copy(stage_v, acc_sh.at[pl.ds(sid * BAGS_PER_W, BAGS_PER_W)])

        def fire_rows(c, b):
            pltpu.async_copy(
                hw_hbm.at[ids_v.at[c]], rows_v.at[b], sem_rows[b]
            )

        def wait_rows(c, b):
            pltpu.make_async_copy(
                hw_hbm.at[ids_v.at[c]], rows_v.at[b], sem_rows[b]
            ).wait()

        def fire_scatter(c, b):
            pltpu.async_copy(rows_v.at[b], acc_sh.at[slot_v.at[c]], sem_sc[b],
                             add=True)

        def wait_scatter(c, b):
            pltpu.make_async_copy(
                rows_v.at[b], acc_sh.at[slot_v.at[c]], sem_sc[b]
            ).wait()

        for b in range(NBUF):
            fire_rows(b, b)

        def group_body(g, carry):
            for u in range(NBUF):
                c = g * NBUF + u
                pu = (u - 1) % NBUF
                wait_rows(c, u)
                fire_scatter(c, u)

                @pl.when(c >= 1)
                def _retire_prev():
                    wait_scatter(c - 1, pu)

                    @pl.when(c - 1 + NBUF < CHUNKS)
                    def _refill():
                        fire_rows(c - 1 + NBUF, pu)

            return carry

        lax.fori_loop(0, CHUNKS // NBUF, group_body, 0)
        wait_scatter(CHUNKS - 1, (CHUNKS - 1) % NBUF)

        pltpu.sync_copy(
            stage_v.at[pl.ds(0, CHUNK)], acc_sh.at[slot_v.at[CHUNKS - 1]],
            add=True,
        )
        plsc.subcore_barrier()
        pltpu.sync_copy(acc_sh.at[pl.ds(sid * BAGS_PER_W, BAGS_PER_W)], stage_v)
        pltpu.sync_copy(stage_v, out_hbm.at[pl.ds(wid * BAGS_PER_W, BAGS_PER_W)])

    return bag_kernel


_IDS_KERNEL = _make_ids_kernel()
_BAG_KERNEL = _make_bag_kernel()


@jax.jit
def kernel(x, hashed_weight, weight_idx):
    xr = x.reshape(NUM_WORKERS, ICHUNKS, ICHUNK)
    slot = jnp.asarray(_SLOT_NP)
    ids = _IDS_KERNEL(xr, weight_idx)
    ids3 = ids.reshape(NUM_WORKERS, CHUNKS, CHUNK)
    return _BAG_KERNEL(ids3, hashed_weight, slot)


# speedup vs baseline: 1.0138x; 1.0117x over previous
import functools

import numpy as np
import jax
import jax.numpy as jnp
from jax import lax
from jax.experimental import pallas as pl
from jax.experimental.pallas import tpu as pltpu
from jax.experimental.pallas import tpu_sc as plsc

NUM_EMB = 1000000
EMB_DIM = 64
HASHED_SIZE = 100000
BATCH = 4096
BAG = 50

NUM_WORKERS = 32
ROWS_PER_W = BATCH * BAG // NUM_WORKERS
CHUNK = 2 * BAG
CHUNKS = ROWS_PER_W // CHUNK
ICHUNK = 128
ICHUNKS = ROWS_PER_W // ICHUNK
BAGS_PER_W = BATCH // NUM_WORKERS
NBUF = 4
RBUF = 8

_SLOT_NP = (
    (np.arange(ROWS_PER_W, dtype=np.int32) // BAG)[None, :]
    + (np.arange(16, dtype=np.int32) * BAGS_PER_W)[:, None]
).reshape(16, CHUNKS, CHUNK)


def _make_ids_kernel():
    mesh = plsc.VectorSubcoreMesh(core_axis_name="c", subcore_axis_name="s")

    @functools.partial(
        pl.kernel,
        mesh=mesh,
        out_type=jax.ShapeDtypeStruct((NUM_WORKERS, ICHUNKS, ICHUNK), jnp.int32),
        scratch_types=[
            pltpu.VMEM((ICHUNKS, ICHUNK), jnp.int32),
            pltpu.VMEM((ICHUNKS, ICHUNK), jnp.int32),
        ] + [pltpu.SemaphoreType.DMA] * NBUF,
        compiler_params=pltpu.CompilerParams(use_tc_tiling_on_sc=False),
    )
    def ids_kernel(x_hbm, wi_hbm, ids_hbm, x_v, ids_v, *sems):
        wid = lax.axis_index("s") * 2 + lax.axis_index("c")

        pltpu.sync_copy(x_hbm.at[wid], x_v)

        def fire(c, b):
            pltpu.async_copy(wi_hbm.at[x_v.at[c]], ids_v.at[c], sems[b])

        def wait(c, b):
            pltpu.make_async_copy(
                wi_hbm.at[x_v.at[c]], ids_v.at[c], sems[b]
            ).wait()

        for b in range(NBUF):
            fire(b, b)

        def body(c, carry):
            for b in range(NBUF):
                cc = c * NBUF + b
                wait(cc, b)

                @pl.when(cc + NBUF < ICHUNKS)
                def _next():
                    fire(cc + NBUF, b)

            return carry

        lax.fori_loop(0, ICHUNKS // NBUF, body, 0)
        for cc in range(ICHUNKS - ICHUNKS % NBUF, ICHUNKS):
            wait(cc, cc % NBUF)

        pltpu.sync_copy(ids_v, ids_hbm.at[wid])

    return ids_kernel


def _make_bag_kernel():
    mesh = plsc.VectorSubcoreMesh(core_axis_name="c", subcore_axis_name="s")

    @functools.partial(
        pl.kernel,
        mesh=mesh,
        out_type=jax.ShapeDtypeStruct((BATCH, EMB_DIM), jnp.float32),
        scratch_types=[
            pltpu.VMEM((CHUNKS, CHUNK), jnp.int32),
            pltpu.VMEM((CHUNKS, CHUNK), jnp.int32),
            pltpu.VMEM((RBUF, CHUNK, EMB_DIM), jnp.float32),
            pltpu.VMEM((BAGS_PER_W, EMB_DIM), jnp.float32),
            pltpu.VMEM_SHARED((16 * BAGS_PER_W, EMB_DIM), jnp.float32),
        ] + [pltpu.SemaphoreType.DMA] * (2 * RBUF),
        compiler_params=pltpu.CompilerParams(use_tc_tiling_on_sc=False),
    )
    def bag_kernel(ids_hbm, hw_hbm, slot_hbm, out_hbm,
                   ids_v, slot_v, rows_v, stage_v, acc_sh, *sems):
        sem_rows = sems[:RBUF]
        sem_sc = sems[RBUF:]
        cid = lax.axis_index("c")
        sid = lax.axis_index("s")
        wid = sid * 2 + cid

        pltpu.sync_copy(ids_hbm.at[wid], ids_v)
        pltpu.sync_copy(slot_hbm.at[sid], slot_v)

        zeros = jnp.zeros((16,), jnp.float32)

        def zero_body(r, carry):
            for k in range(EMB_DIM // 16):
                stage_v[r, pl.ds(k * 16, 16)] = zeros
            return carry

        lax.fori_loop(0, BAGS_PER_W, zero_body, 0)
        pltpu.sync_copy(stage_v, acc_sh.at[pl.ds(sid * BAGS_PER_W, BAGS_PER_W)])

        def fire_rows(c, b):
            pltpu.async_copy(
                hw_hbm.at[ids_v.at[c]], rows_v.at[b], sem_rows[b]
            )

        def wait_rows(c, b):
            pltpu.make_async_copy(
                hw_hbm.at[ids_v.at[c]], rows_v.at[b], sem_rows[b]
            ).wait()

        def fire_scatter(c, b):
            pltpu.async_copy(rows_v.at[b], acc_sh.at[slot_v.at[c]], sem_sc[b],
                             add=True)

        def wait_scatter(c, b):
            pltpu.make_async_copy(
                rows_v.at[b], acc_sh.at[slot_v.at[c]], sem_sc[b]
            ).wait()

        for b in range(RBUF):
            fire_rows(b, b)

        def group_body(g, carry):
            for u in range(RBUF):
                c = g * RBUF + u
                pu = (u - 1) % RBUF
                wait_rows(c, u)
                fire_scatter(c, u)

                @pl.when(c >= 1)
                def _retire_prev():
                    wait_scatter(c - 1, pu)

                    @pl.when(c - 1 + RBUF < CHUNKS)
                    def _refill():
                        fire_rows(c - 1 + RBUF, pu)

            return carry

        lax.fori_loop(0, CHUNKS // RBUF, group_body, 0)
        wait_scatter(CHUNKS - 1, (CHUNKS - 1) % RBUF)

        pltpu.sync_copy(
            stage_v.at[pl.ds(0, CHUNK)], acc_sh.at[slot_v.at[CHUNKS - 1]],
            add=True,
        )
        plsc.subcore_barrier()
        pltpu.sync_copy(acc_sh.at[pl.ds(sid * BAGS_PER_W, BAGS_PER_W)], stage_v)
        pltpu.sync_copy(stage_v, out_hbm.at[pl.ds(wid * BAGS_PER_W, BAGS_PER_W)])

    return bag_kernel


_IDS_KERNEL = _make_ids_kernel()
_BAG_KERNEL = _make_bag_kernel()


@jax.jit
def kernel(x, hashed_weight, weight_idx):
    xr = x.reshape(NUM_WORKERS, ICHUNKS, ICHUNK)
    slot = jnp.asarray(_SLOT_NP)
    ids = _IDS_KERNEL(xr, weight_idx)
    ids3 = ids.reshape(NUM_WORKERS, CHUNKS, CHUNK)
    return _BAG_KERNEL(ids3, hashed_weight, slot)
